# baseline (speedup 1.0000x reference)
import jax
import jax.numpy as jnp
from jax import lax
from jax.experimental import pallas as pl
from jax.experimental.pallas import tpu as pltpu
from jax.experimental.pallas import tpu_sc as plsc

V = 1_000_000
NC = 2
NS = 16
L = 16
NW = NC * NS

CHUNK = 62_976
VPAD = NS * CHUNK
NWAVE = 12
WAVE = CHUNK // NWAVE
SUM_U = 8

B_TOT = 4096 * 200
ROWS = B_TOT // 128
ROWS_W = ROWS // NW
G = 40
NG = ROWS_W // G

_LN2 = 0.6931471805599453
_P4 = (-0.05545986968073571, 0.44050704554227527, -1.4552065437591728,
       2.806994158628966, -1.7367654165499555)


def _log_from_bits(v, ln_s):
    bits = lax.bitcast_convert_type(v, jnp.int32)
    e = lax.shift_right_arithmetic(bits, 23) - 127
    m_bits = lax.bitwise_or(lax.bitwise_and(bits, 0x007FFFFF), 0x3F800000)
    m = lax.bitcast_convert_type(m_bits, jnp.float32)
    p = jnp.float32(_P4[0])
    for c in _P4[1:-1]:
        p = p * m + jnp.float32(c)
    p = p * m + (jnp.float32(_P4[-1]) - ln_s)
    return e.astype(jnp.float32) * jnp.float32(_LN2) + p


def _body(idx_hbm, probs_hbm, out_hbm,
          table_sp, part_sp, buf_a, buf_b, idx_v, vals_v, pvt_v, pall_v,
          sem_a, sem_b, sem_stage, sem_idx, sem_g0, sem_g1, sem_out):
    cid = lax.axis_index("c")
    sid = lax.axis_index("s")
    wid = sid * NC + cid
    obase = wid * ROWS_W

    h_idx = pltpu.async_copy(idx_hbm.at[pl.ds(obase, ROWS_W)], idx_v, sem_idx)

    base = sid * CHUNK
    pltpu.async_copy(probs_hbm.at[pl.ds(base, CHUNK)],
                     table_sp.at[pl.ds(base, CHUNK)], sem_stage).wait()

    plsc.subcore_barrier()

    h_idx.wait()
    sems_g = (sem_g0, sem_g1)

    def fire_group(g):
        s = sems_g[g % 2]

        def fire(r, c):
            row = g * G + r
            pltpu.async_copy(table_sp.at[idx_v.at[row]], vals_v.at[row], s)
            return c
        lax.fori_loop(0, G, fire, 0)

    fire_group(0)
    fire_group(1)

    sbase = base
    sets = (buf_a, buf_b)
    sems = (sem_a, sem_b)

    def fire_wave(w):
        pltpu.async_copy(
            table_sp.at[pl.ds(sbase + w * WAVE, WAVE)],
            sets[w % 2], sems[w % 2])

    fire_wave(0)
    accs = [jnp.zeros((L,), jnp.float32) for _ in range(4)]
    for w in range(NWAVE):
        if w + 1 < NWAVE:
            fire_wave(w + 1)
        bset = sets[w % 2]
        pltpu.make_async_copy(
            probs_hbm.at[pl.ds(0, WAVE)], bset, sems[w % 2]).wait()

        def sum_step(i, a):
            o = i * (SUM_U * L)
            for u in range(SUM_U):
                a = tuple(
                    a[j] + bset[pl.ds(o + u * L, L)] if j == (u % 4) else a[j]
                    for j in range(4))
            return a
        accs = list(lax.fori_loop(0, WAVE // (SUM_U * L), sum_step,
                                  tuple(accs)))

    acc = (accs[0] + accs[1]) + (accs[2] + accs[3])
    pvt_v[...] = acc
    pltpu.sync_copy(pvt_v, part_sp.at[sid])

    plsc.subcore_barrier()

    pltpu.sync_copy(part_sp, pall_v)
    tot = pall_v[0]
    for t in range(1, NS):
        tot = tot + pall_v[t]
    s_scalar = tot[0]
    for i in range(1, L):
        s_scalar = s_scalar + tot[i]
    ln_s = _log_from_bits(jnp.full((L,), s_scalar, jnp.float32),
                          jnp.zeros((L,), jnp.float32))

    for g in range(NG):
        pltpu.make_async_copy(
            out_hbm.at[pl.ds(obase + g * G, G)],
            vals_v.at[pl.ds(g * G, G)], sems_g[g % 2]).wait()
        if g + 2 < NG:
            fire_group(g + 2)

        def log_row(r, carry):
            row = g * G + r
            for c in range(128 // L):
                v = vals_v[row, pl.ds(c * L, L)]
                vals_v[row, pl.ds(c * L, L)] = _log_from_bits(v, ln_s)
            return carry
        lax.fori_loop(0, G, log_row, 0)

        pltpu.async_copy(vals_v.at[pl.ds(g * G, G)],
                         out_hbm.at[pl.ds(obase + g * G, G)], sem_out)

    pltpu.make_async_copy(out_hbm.at[pl.ds(obase, ROWS_W)],
                          vals_v, sem_out).wait()


@jax.jit
def kernel(indices, probs):
    idx2d = indices.reshape(ROWS, 128)
    probs_pad = jnp.pad(probs, (0, VPAD - V))
    mesh = plsc.VectorSubcoreMesh(core_axis_name="c", subcore_axis_name="s")
    fn = pl.kernel(
        _body,
        out_type=jax.ShapeDtypeStruct((ROWS, 128), jnp.float32),
        mesh=mesh,
        scratch_types=[
            pltpu.VMEM_SHARED((VPAD,), jnp.float32),
            pltpu.VMEM_SHARED((NS, L), jnp.float32),
            pltpu.VMEM((WAVE,), jnp.float32),
            pltpu.VMEM((WAVE,), jnp.float32),
            pltpu.VMEM((ROWS_W, 128), jnp.int32),
            pltpu.VMEM((ROWS_W, 128), jnp.float32),
            pltpu.VMEM((L,), jnp.float32),
            pltpu.VMEM((NS, L), jnp.float32),
        ] + [pltpu.SemaphoreType.DMA] * 7,
    )
    out = fn(idx2d, probs_pad)
    return out.reshape(4096, 200)
